# baseline (speedup 1.0000x reference)
import math

import jax
import jax.numpy as jnp
from jax.experimental import pallas as pl
from jax.experimental.pallas import tpu as pltpu


B, S, H, NH = 2, 2048, 2048, 16
HD = H // NH
SCALE = 1.0 / math.sqrt(HD)
LOG2E = math.log2(math.e)

QKV_NG = 4
ATT_HG = 2
ATT_BQ = 256
OUT_MT = 512


def _qkv_kernel(x_ref, w_ref, o_ref):
    acc = jnp.dot(x_ref[0], w_ref[...], preferred_element_type=jnp.float32)
    acc = acc.astype(jnp.bfloat16)
    for j in range(QKV_NG):
        o_ref[0, 0, j] = acc[:, j * HD:(j + 1) * HD]


def _attn_kernel(q_ref, k_ref, v_ref, pb_ref, w_ref, ctx_ref):
    for h in range(ATT_HG):
        pb2 = pb_ref[h] * LOG2E
        for b in range(B):
            q = (q_ref[0, b, h].astype(jnp.float32)
                 * (SCALE * LOG2E)).astype(jnp.bfloat16)
            k = k_ref[0, b, h]
            s = jax.lax.dot_general(q, k, (((1,), (1,)), ((), ())),
                                    preferred_element_type=jnp.float32)
            p = jnp.exp2(s + pb2)
            w = p * (1.0 / jnp.sum(p, axis=-1, keepdims=True))
            w_ref[b, h] = w
            ctx = jnp.dot(w.astype(jnp.bfloat16), v_ref[0, b, h],
                          preferred_element_type=jnp.float32)
            ctx_ref[b, :, h * HD:(h + 1) * HD] = ctx.astype(jnp.bfloat16)


def _out_kernel(x_ref, w_ref, o_ref):
    o_ref[0] = jnp.dot(x_ref[0], w_ref[...], preferred_element_type=jnp.float32)


def kernel(hidden_states, position_bias, W_qkv, W_out):
    f32 = jnp.float32
    bf16 = jnp.bfloat16

    hs16 = hidden_states.astype(bf16)
    wqkv16 = W_qkv.astype(bf16)
    wout16 = W_out.astype(bf16)

    n_col = 3 * NH // QKV_NG
    qkv = pl.pallas_call(
        _qkv_kernel,
        grid=(B, n_col),
        in_specs=[
            pl.BlockSpec((1, S, H), lambda b, n: (b, 0, 0)),
            pl.BlockSpec((H, QKV_NG * HD), lambda b, n: (0, n)),
        ],
        out_specs=pl.BlockSpec(
            (1, 1, QKV_NG, S, HD),
            lambda b, n: (n * QKV_NG // NH, b, n % (NH // QKV_NG), 0, 0)),
        out_shape=jax.ShapeDtypeStruct((3, B, NH, S, HD), bf16),
        compiler_params=pltpu.CompilerParams(
            dimension_semantics=("arbitrary", "arbitrary")),
    )(hs16, wqkv16)

    n_hg = NH // ATT_HG
    n_q = S // ATT_BQ
    weights, context = pl.pallas_call(
        _attn_kernel,
        grid=(n_hg, n_q),
        in_specs=[
            pl.BlockSpec((1, B, ATT_HG, ATT_BQ, HD),
                         lambda g, q: (0, 0, g, q, 0)),
            pl.BlockSpec((1, B, ATT_HG, S, HD),
                         lambda g, q: (1, 0, g, 0, 0)),
            pl.BlockSpec((1, B, ATT_HG, S, HD),
                         lambda g, q: (2, 0, g, 0, 0)),
            pl.BlockSpec((ATT_HG, ATT_BQ, S), lambda g, q: (g, q, 0)),
        ],
        out_specs=[
            pl.BlockSpec((B, ATT_HG, ATT_BQ, S), lambda g, q: (0, g, q, 0)),
            pl.BlockSpec((B, ATT_BQ, ATT_HG * HD), lambda g, q: (0, q, g)),
        ],
        out_shape=[
            jax.ShapeDtypeStruct((B, NH, S, S), f32),
            jax.ShapeDtypeStruct((B, S, H), bf16),
        ],
        compiler_params=pltpu.CompilerParams(
            dimension_semantics=("arbitrary", "arbitrary")),
    )(qkv, qkv, qkv, position_bias)

    attn_output = pl.pallas_call(
        _out_kernel,
        grid=(B, S // OUT_MT),
        in_specs=[
            pl.BlockSpec((1, OUT_MT, H), lambda b, m: (b, m, 0)),
            pl.BlockSpec((H, H), lambda b, m: (0, 0)),
        ],
        out_specs=pl.BlockSpec((1, OUT_MT, H), lambda b, m: (b, m, 0)),
        out_shape=jax.ShapeDtypeStruct((B, S, H), f32),
        compiler_params=pltpu.CompilerParams(
            dimension_semantics=("arbitrary", "arbitrary")),
    )(context, wout16)

    return attn_output, weights


# speedup vs baseline: 1.0476x; 1.0476x over previous
import math

import jax
import jax.numpy as jnp
from jax.experimental import pallas as pl
from jax.experimental.pallas import tpu as pltpu


B, S, H, NH = 2, 2048, 2048, 16
HD = H // NH
SCALE = 1.0 / math.sqrt(HD)
LOG2E = math.log2(math.e)

QKV_NG = 4
ATT_HG = 1
ATT_BQ = 512
OUT_MT = 512


def _qkv_kernel(x_ref, w_ref, o_ref):
    acc = jnp.dot(x_ref[0], w_ref[...], preferred_element_type=jnp.float32)
    acc = acc.astype(jnp.bfloat16)
    for j in range(QKV_NG):
        o_ref[0, 0, j] = acc[:, j * HD:(j + 1) * HD]


def _attn_kernel(q_ref, k_ref, v_ref, pb_ref, w_ref, ctx_ref):
    for h in range(ATT_HG):
        for b in range(B):
            q = (q_ref[0, b, h].astype(jnp.float32)
                 * (SCALE * LOG2E)).astype(jnp.bfloat16)
            k = k_ref[0, b, h]
            s = jax.lax.dot_general(q, k, (((1,), (1,)), ((), ())),
                                    preferred_element_type=jnp.float32)
            p = jnp.exp2(s + pb_ref[h] * LOG2E)
            w = p * (1.0 / jnp.sum(p, axis=-1, keepdims=True))
            w_ref[b, h] = w
            ctx = jnp.dot(w.astype(jnp.bfloat16), v_ref[0, b, h],
                          preferred_element_type=jnp.float32)
            ctx_ref[b, :, h * HD:(h + 1) * HD] = ctx


def _out_kernel(x_ref, w_ref, o_ref):
    o_ref[0] = jnp.dot(x_ref[0], w_ref[...], preferred_element_type=jnp.float32)


def kernel(hidden_states, position_bias, W_qkv, W_out):
    f32 = jnp.float32
    bf16 = jnp.bfloat16

    n_col = 3 * NH // QKV_NG
    qkv = pl.pallas_call(
        _qkv_kernel,
        grid=(B, n_col),
        in_specs=[
            pl.BlockSpec((1, S, H), lambda b, n: (b, 0, 0)),
            pl.BlockSpec((H, QKV_NG * HD), lambda b, n: (0, n)),
        ],
        out_specs=pl.BlockSpec(
            (1, 1, QKV_NG, S, HD),
            lambda b, n: (n * QKV_NG // NH, b, n % (NH // QKV_NG), 0, 0)),
        out_shape=jax.ShapeDtypeStruct((3, B, NH, S, HD), bf16),
        compiler_params=pltpu.CompilerParams(
            dimension_semantics=("arbitrary", "arbitrary")),
    )(hidden_states, W_qkv)

    n_hg = NH // ATT_HG
    n_q = S // ATT_BQ
    weights, context = pl.pallas_call(
        _attn_kernel,
        grid=(n_hg, n_q),
        in_specs=[
            pl.BlockSpec((1, B, ATT_HG, ATT_BQ, HD),
                         lambda g, q: (0, 0, g, q, 0)),
            pl.BlockSpec((1, B, ATT_HG, S, HD),
                         lambda g, q: (1, 0, g, 0, 0)),
            pl.BlockSpec((1, B, ATT_HG, S, HD),
                         lambda g, q: (2, 0, g, 0, 0)),
            pl.BlockSpec((ATT_HG, ATT_BQ, S), lambda g, q: (g, q, 0)),
        ],
        out_specs=[
            pl.BlockSpec((B, ATT_HG, ATT_BQ, S), lambda g, q: (0, g, q, 0)),
            pl.BlockSpec((B, ATT_BQ, ATT_HG * HD), lambda g, q: (0, q, g)),
        ],
        out_shape=[
            jax.ShapeDtypeStruct((B, NH, S, S), f32),
            jax.ShapeDtypeStruct((B, S, H), f32),
        ],
        compiler_params=pltpu.CompilerParams(
            dimension_semantics=("arbitrary", "arbitrary")),
    )(qkv, qkv, qkv, position_bias)

    attn_output = pl.pallas_call(
        _out_kernel,
        grid=(B, S // OUT_MT),
        in_specs=[
            pl.BlockSpec((1, OUT_MT, H), lambda b, m: (b, m, 0)),
            pl.BlockSpec((H, H), lambda b, m: (0, 0)),
        ],
        out_specs=pl.BlockSpec((1, OUT_MT, H), lambda b, m: (b, m, 0)),
        out_shape=jax.ShapeDtypeStruct((B, S, H), f32),
        compiler_params=pltpu.CompilerParams(
            dimension_semantics=("arbitrary", "arbitrary")),
    )(context, W_out)

    return attn_output, weights


# speedup vs baseline: 1.0798x; 1.0307x over previous
import math

import jax
import jax.numpy as jnp
from jax.experimental import pallas as pl
from jax.experimental.pallas import tpu as pltpu


B, S, H, NH = 2, 2048, 2048, 16
HD = H // NH
SCALE = 1.0 / math.sqrt(HD)
LOG2E = math.log2(math.e)

QKV_NG = 4
ATT_HG = 2
ATT_BQ = 256
OUT_MT = 512


def _qkv_kernel(x_ref, w_ref, o_ref):
    acc = jnp.dot(x_ref[0], w_ref[...], preferred_element_type=jnp.float32)
    acc = acc.astype(jnp.bfloat16)
    for j in range(QKV_NG):
        o_ref[0, 0, j] = acc[:, j * HD:(j + 1) * HD]


def _attn_kernel(q_ref, k_ref, v_ref, pb_ref, w_ref, ctx_ref):
    for h in range(ATT_HG):
        for b in range(B):
            q = (q_ref[0, b, h].astype(jnp.float32)
                 * (SCALE * LOG2E)).astype(jnp.bfloat16)
            k = k_ref[0, b, h]
            s = jax.lax.dot_general(q, k, (((1,), (1,)), ((), ())),
                                    preferred_element_type=jnp.float32)
            p = jnp.exp2(s + pb_ref[h] * LOG2E)
            w = p * (1.0 / jnp.sum(p, axis=-1, keepdims=True))
            w_ref[b, h] = w
            ctx = jnp.dot(w.astype(jnp.bfloat16), v_ref[0, b, h],
                          preferred_element_type=jnp.float32)
            ctx_ref[b, :, h * HD:(h + 1) * HD] = ctx


def _out_kernel(x_ref, w_ref, o_ref):
    o_ref[0] = jnp.dot(x_ref[0], w_ref[...], preferred_element_type=jnp.float32)


def kernel(hidden_states, position_bias, W_qkv, W_out):
    f32 = jnp.float32
    bf16 = jnp.bfloat16

    n_col = 3 * NH // QKV_NG
    qkv = pl.pallas_call(
        _qkv_kernel,
        grid=(B, n_col),
        in_specs=[
            pl.BlockSpec((1, S, H), lambda b, n: (b, 0, 0)),
            pl.BlockSpec((H, QKV_NG * HD), lambda b, n: (0, n)),
        ],
        out_specs=pl.BlockSpec(
            (1, 1, QKV_NG, S, HD),
            lambda b, n: (n * QKV_NG // NH, b, n % (NH // QKV_NG), 0, 0)),
        out_shape=jax.ShapeDtypeStruct((3, B, NH, S, HD), bf16),
        compiler_params=pltpu.CompilerParams(
            dimension_semantics=("arbitrary", "arbitrary")),
    )(hidden_states, W_qkv)

    n_hg = NH // ATT_HG
    n_q = S // ATT_BQ
    weights, context = pl.pallas_call(
        _attn_kernel,
        grid=(n_hg, n_q),
        in_specs=[
            pl.BlockSpec((1, B, ATT_HG, ATT_BQ, HD),
                         lambda g, q: (0, 0, g, q, 0)),
            pl.BlockSpec((1, B, ATT_HG, S, HD),
                         lambda g, q: (1, 0, g, 0, 0)),
            pl.BlockSpec((1, B, ATT_HG, S, HD),
                         lambda g, q: (2, 0, g, 0, 0)),
            pl.BlockSpec((ATT_HG, ATT_BQ, S), lambda g, q: (g, q, 0)),
        ],
        out_specs=[
            pl.BlockSpec((B, ATT_HG, ATT_BQ, S), lambda g, q: (0, g, q, 0)),
            pl.BlockSpec((B, ATT_BQ, ATT_HG * HD), lambda g, q: (0, q, g)),
        ],
        out_shape=[
            jax.ShapeDtypeStruct((B, NH, S, S), f32),
            jax.ShapeDtypeStruct((B, S, H), f32),
        ],
        compiler_params=pltpu.CompilerParams(
            dimension_semantics=("arbitrary", "arbitrary")),
    )(qkv, qkv, qkv, position_bias)

    attn_output = pl.pallas_call(
        _out_kernel,
        grid=(B, S // OUT_MT),
        in_specs=[
            pl.BlockSpec((1, OUT_MT, H), lambda b, m: (b, m, 0)),
            pl.BlockSpec((H, H), lambda b, m: (0, 0)),
        ],
        out_specs=pl.BlockSpec((1, OUT_MT, H), lambda b, m: (b, m, 0)),
        out_shape=jax.ShapeDtypeStruct((B, S, H), f32),
        compiler_params=pltpu.CompilerParams(
            dimension_semantics=("arbitrary", "arbitrary")),
    )(context, W_out)

    return attn_output, weights


# speedup vs baseline: 1.0860x; 1.0058x over previous
import math

import jax
import jax.numpy as jnp
from jax.experimental import pallas as pl
from jax.experimental.pallas import tpu as pltpu


B, S, H, NH = 2, 2048, 2048, 16
HD = H // NH
SCALE = 1.0 / math.sqrt(HD)
LOG2E = math.log2(math.e)

QKV_NG = 4
ATT_HG = 2
ATT_BQ = 256
OUT_MT = 512


def _qkv_kernel(x_ref, w_ref, o_ref):
    acc = jnp.dot(x_ref[0], w_ref[...], preferred_element_type=jnp.float32)
    acc = acc.astype(jnp.bfloat16)
    for j in range(QKV_NG):
        o_ref[0, 0, j] = acc[:, j * HD:(j + 1) * HD]


def _attn_kernel(q_ref, k_ref, v_ref, pb_ref, w_ref, ctx_ref):
    for h in range(ATT_HG):
        pb2 = pb_ref[h] * LOG2E
        for b in range(B):
            q = (q_ref[0, b, h].astype(jnp.float32)
                 * (SCALE * LOG2E)).astype(jnp.bfloat16)
            k = k_ref[0, b, h]
            s = jax.lax.dot_general(q, k, (((1,), (1,)), ((), ())),
                                    preferred_element_type=jnp.float32)
            p = jnp.exp2(s + pb2)
            w = p * (1.0 / jnp.sum(p, axis=-1, keepdims=True))
            w_ref[b, h] = w
            ctx = jnp.dot(w.astype(jnp.bfloat16), v_ref[0, b, h],
                          preferred_element_type=jnp.float32)
            ctx_ref[b, :, h * HD:(h + 1) * HD] = ctx


def _out_kernel(x_ref, w_ref, o_ref):
    o_ref[0] = jnp.dot(x_ref[0], w_ref[...], preferred_element_type=jnp.float32)


def kernel(hidden_states, position_bias, W_qkv, W_out):
    f32 = jnp.float32
    bf16 = jnp.bfloat16

    n_col = 3 * NH // QKV_NG
    qkv = pl.pallas_call(
        _qkv_kernel,
        grid=(B, n_col),
        in_specs=[
            pl.BlockSpec((1, S, H), lambda b, n: (b, 0, 0)),
            pl.BlockSpec((H, QKV_NG * HD), lambda b, n: (0, n)),
        ],
        out_specs=pl.BlockSpec(
            (1, 1, QKV_NG, S, HD),
            lambda b, n: (n * QKV_NG // NH, b, n % (NH // QKV_NG), 0, 0)),
        out_shape=jax.ShapeDtypeStruct((3, B, NH, S, HD), bf16),
        compiler_params=pltpu.CompilerParams(
            dimension_semantics=("arbitrary", "arbitrary")),
    )(hidden_states, W_qkv)

    n_hg = NH // ATT_HG
    n_q = S // ATT_BQ
    weights, context = pl.pallas_call(
        _attn_kernel,
        grid=(n_hg, n_q),
        in_specs=[
            pl.BlockSpec((1, B, ATT_HG, ATT_BQ, HD),
                         lambda g, q: (0, 0, g, q, 0)),
            pl.BlockSpec((1, B, ATT_HG, S, HD),
                         lambda g, q: (1, 0, g, 0, 0)),
            pl.BlockSpec((1, B, ATT_HG, S, HD),
                         lambda g, q: (2, 0, g, 0, 0)),
            pl.BlockSpec((ATT_HG, ATT_BQ, S), lambda g, q: (g, q, 0)),
        ],
        out_specs=[
            pl.BlockSpec((B, ATT_HG, ATT_BQ, S), lambda g, q: (0, g, q, 0)),
            pl.BlockSpec((B, ATT_BQ, ATT_HG * HD), lambda g, q: (0, q, g)),
        ],
        out_shape=[
            jax.ShapeDtypeStruct((B, NH, S, S), f32),
            jax.ShapeDtypeStruct((B, S, H), f32),
        ],
        compiler_params=pltpu.CompilerParams(
            dimension_semantics=("arbitrary", "arbitrary")),
    )(qkv, qkv, qkv, position_bias)

    attn_output = pl.pallas_call(
        _out_kernel,
        grid=(B, S // OUT_MT),
        in_specs=[
            pl.BlockSpec((1, OUT_MT, H), lambda b, m: (b, m, 0)),
            pl.BlockSpec((H, H), lambda b, m: (0, 0)),
        ],
        out_specs=pl.BlockSpec((1, OUT_MT, H), lambda b, m: (b, m, 0)),
        out_shape=jax.ShapeDtypeStruct((B, S, H), f32),
        compiler_params=pltpu.CompilerParams(
            dimension_semantics=("arbitrary", "arbitrary")),
    )(context, W_out)

    return attn_output, weights
